# Initial kernel scaffold; baseline (speedup 1.0000x reference)
import jax
import jax.numpy as jnp
from jax.experimental import pallas as pl


def kernel(x, edge_index, W1, b1, W2, b2, W3, b3):
    raise NotImplementedError("write your pallas kernel here")


# speedup vs baseline: 7.9732x; 7.9732x over previous
import functools

import jax
import jax.numpy as jnp
from jax import lax
from jax.experimental import pallas as pl
from jax.experimental.pallas import tpu as pltpu
from jax.experimental.pallas import tpu_sc as plsc

N_NODES = 10000
N_EDGES = 320000
D_FEAT = 128
N_CLASSES = 16

NC, NS = 2, 16
NW = NC * NS
EDGE_B = 128
N_BATCH = 80
E_W = EDGE_B * N_BATCH
E_PAD = E_W * NW
ACC_ROWS = 10240
ZCH = ACC_ROWS // NS
ROWS_OUT = ACC_ROWS // NS


@functools.cache
def _make_agg(D):
    mesh = plsc.VectorSubcoreMesh(core_axis_name="c", subcore_axis_name="s",
                                  num_cores=NC, num_subcores=NS)

    def body(table, src_idx, dst_idx, zrows, out, s_v, d_v, rows_v, acc, sem):
        cid = lax.axis_index("c")
        sid = lax.axis_index("s")
        wid = sid * NC + cid
        pltpu.sync_copy(zrows, acc.at[pl.ds(sid * ZCH, ZCH)])
        plsc.subcore_barrier()
        base0 = wid * E_W

        def step(i, carry):
            base = base0 + i * EDGE_B
            pltpu.sync_copy(src_idx.at[pl.ds(base, EDGE_B)], s_v)
            pltpu.sync_copy(dst_idx.at[pl.ds(base, EDGE_B)], d_v)
            pltpu.async_copy(table.at[s_v], rows_v, sem).wait()
            pltpu.sync_copy(rows_v, acc.at[d_v], add=True)
            return carry

        lax.fori_loop(0, N_BATCH, step, 0)
        plsc.subcore_barrier()
        pltpu.sync_copy(acc.at[pl.ds(sid * ROWS_OUT, ROWS_OUT)],
                        out.at[cid, pl.ds(sid * ROWS_OUT, ROWS_OUT)])

    return pl.kernel(
        body,
        out_type=jax.ShapeDtypeStruct((NC, ACC_ROWS, D), jnp.float32),
        mesh=mesh,
        scratch_types=[
            pltpu.VMEM((EDGE_B,), jnp.int32),
            pltpu.VMEM((EDGE_B,), jnp.int32),
            pltpu.VMEM((EDGE_B, D), jnp.float32),
            pltpu.VMEM_SHARED((ACC_ROWS, D), jnp.float32),
            pltpu.SemaphoreType.DMA,
        ],
        compiler_params=pltpu.CompilerParams(use_tc_tiling_on_sc=False),
    )


_ROWS_BLK = 2000
_GRID = N_NODES // _ROWS_BLK


def _tc1_body(pdeg_ref, x_ref, w_ref, h_ref, hs_ref, dis_ref, dinv_ref):
    deg = pdeg_ref[0, :, :1] + pdeg_ref[1, :, :1] + 1.0
    dis = lax.rsqrt(deg)
    dinv = 1.0 / deg
    h = jnp.dot(x_ref[...], w_ref[...], preferred_element_type=jnp.float32)
    h_ref[...] = h
    hs_ref[...] = h * dis
    dis_ref[...] = dis
    dinv_ref[...] = dinv


def _tc1(pdeg, x, w):
    return pl.pallas_call(
        _tc1_body,
        grid=(_GRID,),
        in_specs=[
            pl.BlockSpec((NC, _ROWS_BLK, N_CLASSES), lambda i: (0, i, 0)),
            pl.BlockSpec((_ROWS_BLK, D_FEAT), lambda i: (i, 0)),
            pl.BlockSpec((D_FEAT, D_FEAT), lambda i: (0, 0)),
        ],
        out_specs=[
            pl.BlockSpec((_ROWS_BLK, D_FEAT), lambda i: (i, 0)),
            pl.BlockSpec((_ROWS_BLK, D_FEAT), lambda i: (i, 0)),
            pl.BlockSpec((_ROWS_BLK, 1), lambda i: (i, 0)),
            pl.BlockSpec((_ROWS_BLK, 1), lambda i: (i, 0)),
        ],
        out_shape=[
            jax.ShapeDtypeStruct((N_NODES, D_FEAT), jnp.float32),
            jax.ShapeDtypeStruct((N_NODES, D_FEAT), jnp.float32),
            jax.ShapeDtypeStruct((N_NODES, 1), jnp.float32),
            jax.ShapeDtypeStruct((N_NODES, 1), jnp.float32),
        ],
    )(pdeg, x, w)


def _tcmid_body(p_ref, hprev_ref, dis_ref, dinv_ref, b_ref, w_ref, h_ref, hs_ref):
    dis = dis_ref[...]
    z = (p_ref[0] + p_ref[1]) * dis + hprev_ref[...] * dinv_ref[...] + b_ref[...]
    a = jnp.maximum(z, 0.0)
    h = jnp.dot(a, w_ref[...], preferred_element_type=jnp.float32)
    h_ref[...] = h
    hs_ref[...] = h * dis


def _tcmid(p, hprev, dis, dinv, b, w):
    dout = w.shape[1]
    return pl.pallas_call(
        _tcmid_body,
        grid=(_GRID,),
        in_specs=[
            pl.BlockSpec((NC, _ROWS_BLK, D_FEAT), lambda i: (0, i, 0)),
            pl.BlockSpec((_ROWS_BLK, D_FEAT), lambda i: (i, 0)),
            pl.BlockSpec((_ROWS_BLK, 1), lambda i: (i, 0)),
            pl.BlockSpec((_ROWS_BLK, 1), lambda i: (i, 0)),
            pl.BlockSpec((1, D_FEAT), lambda i: (0, 0)),
            pl.BlockSpec((D_FEAT, dout), lambda i: (0, 0)),
        ],
        out_specs=[
            pl.BlockSpec((_ROWS_BLK, dout), lambda i: (i, 0)),
            pl.BlockSpec((_ROWS_BLK, dout), lambda i: (i, 0)),
        ],
        out_shape=[
            jax.ShapeDtypeStruct((N_NODES, dout), jnp.float32),
            jax.ShapeDtypeStruct((N_NODES, dout), jnp.float32),
        ],
    )(p, hprev, dis, dinv, b, w)


def _tcfin_body(p_ref, h3_ref, dis_ref, dinv_ref, b_ref, out_ref):
    z = (p_ref[0] + p_ref[1]) * dis_ref[...] + h3_ref[...] * dinv_ref[...] + b_ref[...]
    m = jnp.max(z, axis=1, keepdims=True)
    lse = jnp.log(jnp.sum(jnp.exp(z - m), axis=1, keepdims=True)) + m
    out_ref[...] = z - lse


def _tcfin(p, h3, dis, dinv, b):
    return pl.pallas_call(
        _tcfin_body,
        grid=(_GRID,),
        in_specs=[
            pl.BlockSpec((NC, _ROWS_BLK, N_CLASSES), lambda i: (0, i, 0)),
            pl.BlockSpec((_ROWS_BLK, N_CLASSES), lambda i: (i, 0)),
            pl.BlockSpec((_ROWS_BLK, 1), lambda i: (i, 0)),
            pl.BlockSpec((_ROWS_BLK, 1), lambda i: (i, 0)),
            pl.BlockSpec((1, N_CLASSES), lambda i: (0, 0)),
        ],
        out_specs=pl.BlockSpec((_ROWS_BLK, N_CLASSES), lambda i: (i, 0)),
        out_shape=jax.ShapeDtypeStruct((N_NODES, N_CLASSES), jnp.float32),
    )(p, h3, dis, dinv, b)


def kernel(x, edge_index, W1, b1, W2, b2, W3, b3):
    src = edge_index[0].astype(jnp.int32)
    dst = edge_index[1].astype(jnp.int32)
    pad = E_PAD - N_EDGES
    src_pad = jnp.concatenate([src, jnp.zeros((pad,), jnp.int32)])
    dst_pad = jnp.concatenate([dst, jnp.full((pad,), N_NODES, jnp.int32)])
    ones16 = jnp.ones((N_NODES, N_CLASSES), jnp.float32)
    z128 = jnp.zeros((ZCH, D_FEAT), jnp.float32)
    z16 = jnp.zeros((ZCH, N_CLASSES), jnp.float32)

    agg128 = _make_agg(D_FEAT)
    agg16 = _make_agg(N_CLASSES)
    pdeg = agg16(ones16, src_pad, dst_pad, z16)
    h1, hs1, dis, dinv = _tc1(pdeg, x, W1)
    p1 = agg128(hs1, src_pad, dst_pad, z128)
    h2, hs2 = _tcmid(p1, h1, dis, dinv, b1.reshape(1, -1), W2)
    p2 = agg128(hs2, src_pad, dst_pad, z128)
    h3, hs3 = _tcmid(p2, h2, dis, dinv, b2.reshape(1, -1), W3)
    p3 = agg16(hs3, src_pad, dst_pad, z16)
    return _tcfin(p3, h3, dis, dinv, b3.reshape(1, -1))


# speedup vs baseline: 11.0254x; 1.3828x over previous
import functools

import jax
import jax.numpy as jnp
from jax import lax
from jax.experimental import pallas as pl
from jax.experimental.pallas import tpu as pltpu
from jax.experimental.pallas import tpu_sc as plsc

N_NODES = 10000
N_EDGES = 320000
D_FEAT = 128
N_CLASSES = 16

NC, NS = 2, 16
NW = NC * NS
EDGE_B = 128
N_BATCH = 80
CHUNK = 16
E_W = EDGE_B * N_BATCH
E_PAD = E_W * NW
ACC_ROWS = 10240
ZCH = ACC_ROWS // NS
ROWS_OUT = ACC_ROWS // NS


@functools.cache
def _make_agg(D):
    mesh = plsc.VectorSubcoreMesh(core_axis_name="c", subcore_axis_name="s",
                                  num_cores=NC, num_subcores=NS)

    def body(table, src_idx, dst_idx, zrows, out, s_v, d_v, r0, r1, acc,
             sem0, sem1):
        cid = lax.axis_index("c")
        sid = lax.axis_index("s")
        wid = sid * NC + cid
        pltpu.sync_copy(zrows, acc.at[pl.ds(sid * ZCH, ZCH)])
        plsc.subcore_barrier()

        rbuf = (r0, r1)
        sem = (sem0, sem1)

        def chunk_step(c, carry):
            row0 = wid * N_BATCH + c * CHUNK
            pltpu.sync_copy(src_idx.at[pl.ds(row0, CHUNK)], s_v)
            pltpu.sync_copy(dst_idx.at[pl.ds(row0, CHUNK)], d_v)
            pltpu.async_copy(table.at[s_v.at[0]], r0, sem0)
            for b in range(CHUNK):
                if b + 1 < CHUNK:
                    pltpu.async_copy(table.at[s_v.at[b + 1]],
                                     rbuf[(b + 1) % 2], sem[(b + 1) % 2])
                pltpu.make_async_copy(table.at[s_v.at[b]],
                                      rbuf[b % 2], sem[b % 2]).wait()
                pltpu.sync_copy(rbuf[b % 2], acc.at[d_v.at[b]], add=True)
            return carry

        lax.fori_loop(0, N_BATCH // CHUNK, chunk_step, 0)
        plsc.subcore_barrier()
        pltpu.sync_copy(acc.at[pl.ds(sid * ROWS_OUT, ROWS_OUT)],
                        out.at[cid, pl.ds(sid * ROWS_OUT, ROWS_OUT)])

    return pl.kernel(
        body,
        out_type=jax.ShapeDtypeStruct((NC, ACC_ROWS, D), jnp.float32),
        mesh=mesh,
        scratch_types=[
            pltpu.VMEM((CHUNK, EDGE_B), jnp.int32),
            pltpu.VMEM((CHUNK, EDGE_B), jnp.int32),
            pltpu.VMEM((EDGE_B, D), jnp.float32),
            pltpu.VMEM((EDGE_B, D), jnp.float32),
            pltpu.VMEM_SHARED((ACC_ROWS, D), jnp.float32),
            pltpu.SemaphoreType.DMA,
            pltpu.SemaphoreType.DMA,
        ],
        compiler_params=pltpu.CompilerParams(use_tc_tiling_on_sc=False),
    )


_ROWS_BLK = 2000
_GRID = N_NODES // _ROWS_BLK


def _tc1_body(pdeg_ref, x_ref, w_ref, h_ref, hs_ref, dis_ref, dinv_ref):
    deg = pdeg_ref[0, :, :1] + pdeg_ref[1, :, :1] + 1.0
    dis = lax.rsqrt(deg)
    dinv = 1.0 / deg
    h = jnp.dot(x_ref[...], w_ref[...], preferred_element_type=jnp.float32)
    h_ref[...] = h
    hs_ref[...] = h * dis
    dis_ref[...] = dis
    dinv_ref[...] = dinv


def _tc1(pdeg, x, w):
    return pl.pallas_call(
        _tc1_body,
        grid=(_GRID,),
        in_specs=[
            pl.BlockSpec((NC, _ROWS_BLK, N_CLASSES), lambda i: (0, i, 0)),
            pl.BlockSpec((_ROWS_BLK, D_FEAT), lambda i: (i, 0)),
            pl.BlockSpec((D_FEAT, D_FEAT), lambda i: (0, 0)),
        ],
        out_specs=[
            pl.BlockSpec((_ROWS_BLK, D_FEAT), lambda i: (i, 0)),
            pl.BlockSpec((_ROWS_BLK, D_FEAT), lambda i: (i, 0)),
            pl.BlockSpec((_ROWS_BLK, 1), lambda i: (i, 0)),
            pl.BlockSpec((_ROWS_BLK, 1), lambda i: (i, 0)),
        ],
        out_shape=[
            jax.ShapeDtypeStruct((N_NODES, D_FEAT), jnp.float32),
            jax.ShapeDtypeStruct((N_NODES, D_FEAT), jnp.float32),
            jax.ShapeDtypeStruct((N_NODES, 1), jnp.float32),
            jax.ShapeDtypeStruct((N_NODES, 1), jnp.float32),
        ],
    )(pdeg, x, w)


def _tcmid_body(p_ref, hprev_ref, dis_ref, dinv_ref, b_ref, w_ref, h_ref, hs_ref):
    dis = dis_ref[...]
    z = (p_ref[0] + p_ref[1]) * dis + hprev_ref[...] * dinv_ref[...] + b_ref[...]
    a = jnp.maximum(z, 0.0)
    h = jnp.dot(a, w_ref[...], preferred_element_type=jnp.float32)
    h_ref[...] = h
    hs_ref[...] = h * dis


def _tcmid(p, hprev, dis, dinv, b, w):
    dout = w.shape[1]
    return pl.pallas_call(
        _tcmid_body,
        grid=(_GRID,),
        in_specs=[
            pl.BlockSpec((NC, _ROWS_BLK, D_FEAT), lambda i: (0, i, 0)),
            pl.BlockSpec((_ROWS_BLK, D_FEAT), lambda i: (i, 0)),
            pl.BlockSpec((_ROWS_BLK, 1), lambda i: (i, 0)),
            pl.BlockSpec((_ROWS_BLK, 1), lambda i: (i, 0)),
            pl.BlockSpec((1, D_FEAT), lambda i: (0, 0)),
            pl.BlockSpec((D_FEAT, dout), lambda i: (0, 0)),
        ],
        out_specs=[
            pl.BlockSpec((_ROWS_BLK, dout), lambda i: (i, 0)),
            pl.BlockSpec((_ROWS_BLK, dout), lambda i: (i, 0)),
        ],
        out_shape=[
            jax.ShapeDtypeStruct((N_NODES, dout), jnp.float32),
            jax.ShapeDtypeStruct((N_NODES, dout), jnp.float32),
        ],
    )(p, hprev, dis, dinv, b, w)


def _tcfin_body(p_ref, h3_ref, dis_ref, dinv_ref, b_ref, out_ref):
    z = (p_ref[0] + p_ref[1]) * dis_ref[...] + h3_ref[...] * dinv_ref[...] + b_ref[...]
    m = jnp.max(z, axis=1, keepdims=True)
    lse = jnp.log(jnp.sum(jnp.exp(z - m), axis=1, keepdims=True)) + m
    out_ref[...] = z - lse


def _tcfin(p, h3, dis, dinv, b):
    return pl.pallas_call(
        _tcfin_body,
        grid=(_GRID,),
        in_specs=[
            pl.BlockSpec((NC, _ROWS_BLK, N_CLASSES), lambda i: (0, i, 0)),
            pl.BlockSpec((_ROWS_BLK, N_CLASSES), lambda i: (i, 0)),
            pl.BlockSpec((_ROWS_BLK, 1), lambda i: (i, 0)),
            pl.BlockSpec((_ROWS_BLK, 1), lambda i: (i, 0)),
            pl.BlockSpec((1, N_CLASSES), lambda i: (0, 0)),
        ],
        out_specs=pl.BlockSpec((_ROWS_BLK, N_CLASSES), lambda i: (i, 0)),
        out_shape=jax.ShapeDtypeStruct((N_NODES, N_CLASSES), jnp.float32),
    )(p, h3, dis, dinv, b)


def kernel(x, edge_index, W1, b1, W2, b2, W3, b3):
    src = edge_index[0].astype(jnp.int32)
    dst = edge_index[1].astype(jnp.int32)
    pad = E_PAD - N_EDGES
    src_pad = jnp.concatenate(
        [src, jnp.zeros((pad,), jnp.int32)]).reshape(NW * N_BATCH, EDGE_B)
    dst_pad = jnp.concatenate(
        [dst, jnp.full((pad,), N_NODES, jnp.int32)]).reshape(NW * N_BATCH, EDGE_B)
    ones16 = jnp.ones((N_NODES, N_CLASSES), jnp.float32)
    z128 = jnp.zeros((ZCH, D_FEAT), jnp.float32)
    z16 = jnp.zeros((ZCH, N_CLASSES), jnp.float32)

    agg128 = _make_agg(D_FEAT)
    agg16 = _make_agg(N_CLASSES)
    pdeg = agg16(ones16, src_pad, dst_pad, z16)
    h1, hs1, dis, dinv = _tc1(pdeg, x, W1)
    p1 = agg128(hs1, src_pad, dst_pad, z128)
    h2, hs2 = _tcmid(p1, h1, dis, dinv, b1.reshape(1, -1), W2)
    p2 = agg128(hs2, src_pad, dst_pad, z128)
    h3, hs3 = _tcmid(p2, h2, dis, dinv, b2.reshape(1, -1), W3)
    p3 = agg16(hs3, src_pad, dst_pad, z16)
    return _tcfin(p3, h3, dis, dinv, b3.reshape(1, -1))


# speedup vs baseline: 11.0815x; 1.0051x over previous
import functools

import jax
import jax.numpy as jnp
from jax import lax
from jax.experimental import pallas as pl
from jax.experimental.pallas import tpu as pltpu
from jax.experimental.pallas import tpu_sc as plsc

N_NODES = 10000
N_EDGES = 320000
D_FEAT = 128
N_CLASSES = 16

NC, NS = 2, 16
NW = NC * NS
EDGE_B = 128
N_BATCH = 80
CHUNK = 16
E_W = EDGE_B * N_BATCH
E_PAD = E_W * NW
ACC_ROWS = 10240
ZCH = ACC_ROWS // NS
ROWS_OUT = ACC_ROWS // NS


@functools.cache
def _make_agg(D):
    mesh = plsc.VectorSubcoreMesh(core_axis_name="c", subcore_axis_name="s",
                                  num_cores=NC, num_subcores=NS)

    def body(table, src_idx, dst_idx, zrows, out, s_v, d_v, r0, r1, acc,
             sem0, sem1):
        cid = lax.axis_index("c")
        sid = lax.axis_index("s")
        wid = sid * NC + cid
        pltpu.sync_copy(zrows, acc.at[pl.ds(sid * ZCH, ZCH)])
        plsc.subcore_barrier()

        rbuf = (r0, r1)
        sem = (sem0, sem1)

        def chunk_step(c, carry):
            row0 = wid * N_BATCH + c * CHUNK
            pltpu.sync_copy(src_idx.at[pl.ds(row0, CHUNK)], s_v)
            pltpu.sync_copy(dst_idx.at[pl.ds(row0, CHUNK)], d_v)
            pltpu.async_copy(table.at[s_v.at[0]], r0, sem0)
            for b in range(CHUNK):
                if b + 1 < CHUNK:
                    pltpu.async_copy(table.at[s_v.at[b + 1]],
                                     rbuf[(b + 1) % 2], sem[(b + 1) % 2])
                pltpu.make_async_copy(table.at[s_v.at[b]],
                                      rbuf[b % 2], sem[b % 2]).wait()
                pltpu.sync_copy(rbuf[b % 2], acc.at[d_v.at[b]], add=True)
            return carry

        lax.fori_loop(0, N_BATCH // CHUNK, chunk_step, 0)
        plsc.subcore_barrier()
        pltpu.sync_copy(acc.at[pl.ds(sid * ROWS_OUT, ROWS_OUT)],
                        out.at[cid, pl.ds(sid * ROWS_OUT, ROWS_OUT)])

    return pl.kernel(
        body,
        out_type=jax.ShapeDtypeStruct((NC, ACC_ROWS, D), jnp.float32),
        mesh=mesh,
        scratch_types=[
            pltpu.VMEM((CHUNK, EDGE_B), jnp.int32),
            pltpu.VMEM((CHUNK, EDGE_B), jnp.int32),
            pltpu.VMEM((EDGE_B, D), jnp.float32),
            pltpu.VMEM((EDGE_B, D), jnp.float32),
            pltpu.VMEM_SHARED((ACC_ROWS, D), jnp.float32),
            pltpu.SemaphoreType.DMA,
            pltpu.SemaphoreType.DMA,
        ],
        compiler_params=pltpu.CompilerParams(use_tc_tiling_on_sc=False),
    )


_ROWS_BLK = 2000
_GRID = N_NODES // _ROWS_BLK


def _tc1_body(pdeg_ref, x_ref, w_ref, h_ref, hs_ref, dis_ref, dinv_ref):
    deg = pdeg_ref[0, :, :1] + pdeg_ref[1, :, :1] + 1.0
    dis = lax.rsqrt(deg)
    dinv = 1.0 / deg
    h = jnp.dot(x_ref[...], w_ref[...], preferred_element_type=jnp.float32)
    h_ref[...] = h
    hs_ref[...] = h * dis
    dis_ref[...] = dis
    dinv_ref[...] = dinv


def _tc1(pdeg, x, w):
    return pl.pallas_call(
        _tc1_body,
        grid=(_GRID,),
        in_specs=[
            pl.BlockSpec((NC, _ROWS_BLK, N_CLASSES), lambda i: (0, i, 0)),
            pl.BlockSpec((_ROWS_BLK, D_FEAT), lambda i: (i, 0)),
            pl.BlockSpec((D_FEAT, D_FEAT), lambda i: (0, 0)),
        ],
        out_specs=[
            pl.BlockSpec((_ROWS_BLK, D_FEAT), lambda i: (i, 0)),
            pl.BlockSpec((_ROWS_BLK, D_FEAT), lambda i: (i, 0)),
            pl.BlockSpec((_ROWS_BLK, 1), lambda i: (i, 0)),
            pl.BlockSpec((_ROWS_BLK, 1), lambda i: (i, 0)),
        ],
        out_shape=[
            jax.ShapeDtypeStruct((N_NODES, D_FEAT), jnp.float32),
            jax.ShapeDtypeStruct((N_NODES, D_FEAT), jnp.float32),
            jax.ShapeDtypeStruct((N_NODES, 1), jnp.float32),
            jax.ShapeDtypeStruct((N_NODES, 1), jnp.float32),
        ],
    )(pdeg, x, w)


def _tcmid_body(p_ref, hprev_ref, dis_ref, dinv_ref, b_ref, w_ref, h_ref, hs_ref):
    dis = dis_ref[...]
    z = (p_ref[0] + p_ref[1]) * dis + hprev_ref[...] * dinv_ref[...] + b_ref[...]
    a = jnp.maximum(z, 0.0)
    h = jnp.dot(a, w_ref[...], preferred_element_type=jnp.float32)
    h_ref[...] = h
    hs_ref[...] = h * dis


def _tcmid(p, hprev, dis, dinv, b, w):
    dout = w.shape[1]
    return pl.pallas_call(
        _tcmid_body,
        grid=(_GRID,),
        in_specs=[
            pl.BlockSpec((NC, _ROWS_BLK, D_FEAT), lambda i: (0, i, 0)),
            pl.BlockSpec((_ROWS_BLK, D_FEAT), lambda i: (i, 0)),
            pl.BlockSpec((_ROWS_BLK, 1), lambda i: (i, 0)),
            pl.BlockSpec((_ROWS_BLK, 1), lambda i: (i, 0)),
            pl.BlockSpec((1, D_FEAT), lambda i: (0, 0)),
            pl.BlockSpec((D_FEAT, dout), lambda i: (0, 0)),
        ],
        out_specs=[
            pl.BlockSpec((_ROWS_BLK, dout), lambda i: (i, 0)),
            pl.BlockSpec((_ROWS_BLK, dout), lambda i: (i, 0)),
        ],
        out_shape=[
            jax.ShapeDtypeStruct((N_NODES, dout), jnp.float32),
            jax.ShapeDtypeStruct((N_NODES, dout), jnp.float32),
        ],
    )(p, hprev, dis, dinv, b, w)


def _tcfin_body(p_ref, h3_ref, dis_ref, dinv_ref, b_ref, out_ref):
    z = (p_ref[0] + p_ref[1]) * dis_ref[...] + h3_ref[...] * dinv_ref[...] + b_ref[...]
    m = jnp.max(z, axis=1, keepdims=True)
    lse = jnp.log(jnp.sum(jnp.exp(z - m), axis=1, keepdims=True)) + m
    out_ref[...] = z - lse


def _tcfin(p, h3, dis, dinv, b):
    return pl.pallas_call(
        _tcfin_body,
        grid=(_GRID,),
        in_specs=[
            pl.BlockSpec((NC, _ROWS_BLK, N_CLASSES), lambda i: (0, i, 0)),
            pl.BlockSpec((_ROWS_BLK, N_CLASSES), lambda i: (i, 0)),
            pl.BlockSpec((_ROWS_BLK, 1), lambda i: (i, 0)),
            pl.BlockSpec((_ROWS_BLK, 1), lambda i: (i, 0)),
            pl.BlockSpec((1, N_CLASSES), lambda i: (0, 0)),
        ],
        out_specs=pl.BlockSpec((_ROWS_BLK, N_CLASSES), lambda i: (i, 0)),
        out_shape=jax.ShapeDtypeStruct((N_NODES, N_CLASSES), jnp.float32),
    )(p, h3, dis, dinv, b)


def kernel(x, edge_index, W1, b1, W2, b2, W3, b3):
    src = edge_index[0].astype(jnp.int32)
    dst = edge_index[1].astype(jnp.int32)
    pad = E_PAD - N_EDGES
    src_pad = jnp.concatenate(
        [src, jnp.zeros((pad,), jnp.int32)]).reshape(NW * N_BATCH, EDGE_B)
    pad_dst = N_NODES + (jnp.arange(pad, dtype=jnp.int32) % EDGE_B)
    dst_pad = jnp.concatenate([dst, pad_dst]).reshape(NW * N_BATCH, EDGE_B)
    ones16 = jnp.ones((N_NODES, N_CLASSES), jnp.float32)
    z128 = jnp.zeros((ZCH, D_FEAT), jnp.float32)
    z16 = jnp.zeros((ZCH, N_CLASSES), jnp.float32)

    agg128 = _make_agg(D_FEAT)
    agg16 = _make_agg(N_CLASSES)
    pdeg = agg16(ones16, src_pad, dst_pad, z16)
    h1, hs1, dis, dinv = _tc1(pdeg, x, W1)
    p1 = agg128(hs1, src_pad, dst_pad, z128)
    h2, hs2 = _tcmid(p1, h1, dis, dinv, b1.reshape(1, -1), W2)
    p2 = agg128(hs2, src_pad, dst_pad, z128)
    h3, hs3 = _tcmid(p2, h2, dis, dinv, b2.reshape(1, -1), W3)
    p3 = agg16(hs3, src_pad, dst_pad, z16)
    return _tcfin(p3, h3, dis, dinv, b3.reshape(1, -1))


# speedup vs baseline: 21.4514x; 1.9358x over previous
import functools

import jax
import jax.numpy as jnp
from jax import lax
from jax.experimental import pallas as pl
from jax.experimental.pallas import tpu as pltpu
from jax.experimental.pallas import tpu_sc as plsc

N_NODES = 10000
N_EDGES = 320000
D_FEAT = 128
N_CLASSES = 16

NC, NS = 2, 16
NW = NC * NS
EDGE_B = 128
N_BATCH = 80
CHUNK = 16
E_W = EDGE_B * N_BATCH
E_PAD = E_W * NW
ACC_ROWS = 10240
ZCH = ACC_ROWS // NS
ROWS_OUT = ACC_ROWS // NS


DH = D_FEAT // NC
E_T = E_PAD // NS
NB_T = E_T // EDGE_B


@functools.cache
def _make_agg(D, split):
    mesh = plsc.VectorSubcoreMesh(core_axis_name="c", subcore_axis_name="s",
                                  num_cores=NC, num_subcores=NS)
    n_batch = NB_T if split else N_BATCH

    def body(table, src_idx, dst_idx, zrows, out, s_v, d_v, r0, r1, tbl, acc,
             sem0, sem1):
        cid = lax.axis_index("c")
        sid = lax.axis_index("s")
        pltpu.sync_copy(zrows, acc.at[pl.ds(sid * ZCH, ZCH)])

        @pl.when(sid == 0)
        def _():
            if split:
                pltpu.sync_copy(table.at[cid], tbl)
            else:
                pltpu.sync_copy(table, tbl)

        plsc.subcore_barrier()

        rbuf = (r0, r1)
        sem = (sem0, sem1)
        first_row = sid * NB_T if split else (sid * NC + cid) * N_BATCH

        def chunk_step(c, carry):
            row0 = first_row + c * CHUNK
            pltpu.sync_copy(src_idx.at[pl.ds(row0, CHUNK)], s_v)
            pltpu.sync_copy(dst_idx.at[pl.ds(row0, CHUNK)], d_v)
            pltpu.async_copy(tbl.at[s_v.at[0]], r0, sem0)
            for b in range(CHUNK):
                if b + 1 < CHUNK:
                    pltpu.async_copy(tbl.at[s_v.at[b + 1]],
                                     rbuf[(b + 1) % 2], sem[(b + 1) % 2])
                pltpu.make_async_copy(tbl.at[s_v.at[b]],
                                      rbuf[b % 2], sem[b % 2]).wait()
                pltpu.sync_copy(rbuf[b % 2], acc.at[d_v.at[b]], add=True)
            return carry

        lax.fori_loop(0, n_batch // CHUNK, chunk_step, 0)
        plsc.subcore_barrier()
        pltpu.sync_copy(acc.at[pl.ds(sid * ROWS_OUT, ROWS_OUT)],
                        out.at[cid, pl.ds(sid * ROWS_OUT, ROWS_OUT)])

    return pl.kernel(
        body,
        out_type=jax.ShapeDtypeStruct((NC, ACC_ROWS, D), jnp.float32),
        mesh=mesh,
        scratch_types=[
            pltpu.VMEM((CHUNK, EDGE_B), jnp.int32),
            pltpu.VMEM((CHUNK, EDGE_B), jnp.int32),
            pltpu.VMEM((EDGE_B, D), jnp.float32),
            pltpu.VMEM((EDGE_B, D), jnp.float32),
            pltpu.VMEM_SHARED((N_NODES, D), jnp.float32),
            pltpu.VMEM_SHARED((ACC_ROWS, D), jnp.float32),
            pltpu.SemaphoreType.DMA,
            pltpu.SemaphoreType.DMA,
        ],
        compiler_params=pltpu.CompilerParams(use_tc_tiling_on_sc=False),
    )


_ROWS_BLK = 2000
_GRID = N_NODES // _ROWS_BLK


def _split_store(hs_ref, hs):
    hs_ref[0] = hs[:, :DH]
    hs_ref[1] = hs[:, DH:]


def _tc1_body(pdeg_ref, x_ref, w_ref, h_ref, hs_ref, dis_ref, dinv_ref):
    deg = pdeg_ref[0, :, :1] + pdeg_ref[1, :, :1] + 1.0
    dis = lax.rsqrt(deg)
    dinv = 1.0 / deg
    h = jnp.dot(x_ref[...], w_ref[...], preferred_element_type=jnp.float32)
    h_ref[...] = h
    _split_store(hs_ref, h * dis)
    dis_ref[...] = dis
    dinv_ref[...] = dinv


def _tc1(pdeg, x, w):
    return pl.pallas_call(
        _tc1_body,
        grid=(_GRID,),
        in_specs=[
            pl.BlockSpec((NC, _ROWS_BLK, N_CLASSES), lambda i: (0, i, 0)),
            pl.BlockSpec((_ROWS_BLK, D_FEAT), lambda i: (i, 0)),
            pl.BlockSpec((D_FEAT, D_FEAT), lambda i: (0, 0)),
        ],
        out_specs=[
            pl.BlockSpec((_ROWS_BLK, D_FEAT), lambda i: (i, 0)),
            pl.BlockSpec((NC, _ROWS_BLK, DH), lambda i: (0, i, 0)),
            pl.BlockSpec((_ROWS_BLK, 1), lambda i: (i, 0)),
            pl.BlockSpec((_ROWS_BLK, 1), lambda i: (i, 0)),
        ],
        out_shape=[
            jax.ShapeDtypeStruct((N_NODES, D_FEAT), jnp.float32),
            jax.ShapeDtypeStruct((NC, N_NODES, DH), jnp.float32),
            jax.ShapeDtypeStruct((N_NODES, 1), jnp.float32),
            jax.ShapeDtypeStruct((N_NODES, 1), jnp.float32),
        ],
    )(pdeg, x, w)


def _tcmid_body(p_ref, hprev_ref, dis_ref, dinv_ref, b_ref, w_ref, h_ref, hs_ref):
    dis = dis_ref[...]
    agg = jnp.concatenate([p_ref[0], p_ref[1]], axis=1)
    z = agg * dis + hprev_ref[...] * dinv_ref[...] + b_ref[...]
    a = jnp.maximum(z, 0.0)
    h = jnp.dot(a, w_ref[...], preferred_element_type=jnp.float32)
    h_ref[...] = h
    if h.shape[1] == D_FEAT:
        _split_store(hs_ref, h * dis)
    else:
        hs_ref[...] = h * dis


def _tcmid(p, hprev, dis, dinv, b, w):
    dout = w.shape[1]
    if dout == D_FEAT:
        hs_spec = pl.BlockSpec((NC, _ROWS_BLK, DH), lambda i: (0, i, 0))
        hs_shape = jax.ShapeDtypeStruct((NC, N_NODES, DH), jnp.float32)
    else:
        hs_spec = pl.BlockSpec((_ROWS_BLK, dout), lambda i: (i, 0))
        hs_shape = jax.ShapeDtypeStruct((N_NODES, dout), jnp.float32)
    return pl.pallas_call(
        _tcmid_body,
        grid=(_GRID,),
        in_specs=[
            pl.BlockSpec((NC, _ROWS_BLK, DH), lambda i: (0, i, 0)),
            pl.BlockSpec((_ROWS_BLK, D_FEAT), lambda i: (i, 0)),
            pl.BlockSpec((_ROWS_BLK, 1), lambda i: (i, 0)),
            pl.BlockSpec((_ROWS_BLK, 1), lambda i: (i, 0)),
            pl.BlockSpec((1, D_FEAT), lambda i: (0, 0)),
            pl.BlockSpec((D_FEAT, dout), lambda i: (0, 0)),
        ],
        out_specs=[
            pl.BlockSpec((_ROWS_BLK, dout), lambda i: (i, 0)),
            hs_spec,
        ],
        out_shape=[
            jax.ShapeDtypeStruct((N_NODES, dout), jnp.float32),
            hs_shape,
        ],
    )(p, hprev, dis, dinv, b, w)


def _tcfin_body(p_ref, h3_ref, dis_ref, dinv_ref, b_ref, out_ref):
    z = (p_ref[0] + p_ref[1]) * dis_ref[...] + h3_ref[...] * dinv_ref[...] + b_ref[...]
    m = jnp.max(z, axis=1, keepdims=True)
    lse = jnp.log(jnp.sum(jnp.exp(z - m), axis=1, keepdims=True)) + m
    out_ref[...] = z - lse


def _tcfin(p, h3, dis, dinv, b):
    return pl.pallas_call(
        _tcfin_body,
        grid=(_GRID,),
        in_specs=[
            pl.BlockSpec((NC, _ROWS_BLK, N_CLASSES), lambda i: (0, i, 0)),
            pl.BlockSpec((_ROWS_BLK, N_CLASSES), lambda i: (i, 0)),
            pl.BlockSpec((_ROWS_BLK, 1), lambda i: (i, 0)),
            pl.BlockSpec((_ROWS_BLK, 1), lambda i: (i, 0)),
            pl.BlockSpec((1, N_CLASSES), lambda i: (0, 0)),
        ],
        out_specs=pl.BlockSpec((_ROWS_BLK, N_CLASSES), lambda i: (i, 0)),
        out_shape=jax.ShapeDtypeStruct((N_NODES, N_CLASSES), jnp.float32),
    )(p, h3, dis, dinv, b)


def kernel(x, edge_index, W1, b1, W2, b2, W3, b3):
    src = edge_index[0].astype(jnp.int32)
    dst = edge_index[1].astype(jnp.int32)
    pad = E_PAD - N_EDGES
    src_pad = jnp.concatenate(
        [src, jnp.zeros((pad,), jnp.int32)]).reshape(NW * N_BATCH, EDGE_B)
    pad_dst = N_NODES + (jnp.arange(pad, dtype=jnp.int32) % EDGE_B)
    dst_pad = jnp.concatenate([dst, pad_dst]).reshape(NW * N_BATCH, EDGE_B)
    ones16 = jnp.ones((N_NODES, N_CLASSES), jnp.float32)
    z64 = jnp.zeros((ZCH, DH), jnp.float32)
    z16 = jnp.zeros((ZCH, N_CLASSES), jnp.float32)

    agg128 = _make_agg(DH, True)
    agg16 = _make_agg(N_CLASSES, False)
    pdeg = agg16(ones16, src_pad, dst_pad, z16)
    h1, hs1, dis, dinv = _tc1(pdeg, x, W1)
    p1 = agg128(hs1, src_pad, dst_pad, z64)
    h2, hs2 = _tcmid(p1, h1, dis, dinv, b1.reshape(1, -1), W2)
    p2 = agg128(hs2, src_pad, dst_pad, z64)
    h3, hs3 = _tcmid(p2, h2, dis, dinv, b2.reshape(1, -1), W3)
    p3 = agg16(hs3, src_pad, dst_pad, z16)
    return _tcfin(p3, h3, dis, dinv, b3.reshape(1, -1))
